# Initial kernel scaffold; baseline (speedup 1.0000x reference)
import jax
import jax.numpy as jnp
from jax import lax
from jax.experimental import pallas as pl
from jax.experimental.pallas import tpu as pltpu
from jax.experimental.pallas import tpu_sc as plsc

SHAPE = (4, 224, 224, 96)
N = 19267584
U = 4816896
NCHUNK = 10
ROUNDS = 5
CHUNK = 1933312
TRASH = 512
ACC = CHUNK + TRASH
NOUT = NCHUNK * CHUNK
NW = 32
PER_TILE = U // NW
P = 18816
NPIECE = PER_TILE // P
GROUPS = P // 16
ZB = 2048
ZSHARE = ACC // 16
ZNB = ZSHARE // ZB
ZREM = ZSHARE - ZNB * ZB
DUMP_TW = CHUNK // 16


def _sc_body(idx_hbm, val_hbm, out_hbm, idx_v, lidx_v, val_v, zero_v):
    c = lax.axis_index("c")
    s = lax.axis_index("s")
    wid = s * 2 + c

    def _acc_rounds(acc):
        def zinit(i, _):
            zero_v[pl.ds(i * 16, 16)] = jnp.zeros((16,), jnp.float32)
            return 0
        lax.fori_loop(0, ZB // 16, zinit, 0)

        lane = lax.iota(jnp.int32, 16)
        trash_vec = jnp.int32(CHUNK) + s * 16 + lane

        for r in range(ROUNDS):
            q = c * ROUNDS + r
            base = q * CHUNK

            def zblk(i, _):
                pltpu.sync_copy(zero_v, acc.at[pl.ds(s * ZSHARE + i * ZB, ZB)])
                return 0
            lax.fori_loop(0, ZNB, zblk, 0)
            pltpu.sync_copy(zero_v.at[pl.ds(0, ZREM)],
                            acc.at[pl.ds(s * ZSHARE + ZNB * ZB, ZREM)])
            plsc.subcore_barrier()

            def piece(p, _):
                off = wid * PER_TILE + p * P
                pltpu.sync_copy(idx_hbm.at[pl.ds(off, P)], idx_v)
                pltpu.sync_copy(val_hbm.at[pl.ds(off, P)], val_v)

                def grp(i, _):
                    iv = idx_v[pl.ds(i * 16, 16)]
                    lidx = iv - base
                    oob = plsc.bitcast(lidx, jnp.uint32) >= jnp.uint32(CHUNK)
                    lidx_v[pl.ds(i * 16, 16)] = jnp.where(oob, trash_vec, lidx)
                    return 0
                lax.fori_loop(0, GROUPS, grp, 0)
                pltpu.sync_copy(val_v, acc.at[lidx_v], add=True)
                return 0
            lax.fori_loop(0, NPIECE, piece, 0)
            plsc.subcore_barrier()

            pltpu.sync_copy(acc.at[pl.ds(s * DUMP_TW, DUMP_TW)],
                            out_hbm.at[pl.ds(base + s * DUMP_TW, DUMP_TW)])
            plsc.subcore_barrier()

    pl.run_scoped(_acc_rounds,
                  plsc.MemoryRef((ACC,), jnp.float32, pltpu.VMEM_SHARED))


@jax.jit
def _unpool_scatter(idx, val):
    mesh = plsc.VectorSubcoreMesh(core_axis_name="c", subcore_axis_name="s")
    f = pl.kernel(
        _sc_body,
        out_type=jax.ShapeDtypeStruct((NOUT,), jnp.float32),
        mesh=mesh,
        scratch_types=[
            pltpu.VMEM((P,), jnp.int32),
            pltpu.VMEM((P,), jnp.int32),
            pltpu.VMEM((P,), jnp.float32),
            pltpu.VMEM((ZB,), jnp.float32),
        ],
    )
    return f(idx, val)


def kernel(inputs, indices, shape_before):
    del shape_before
    idx = indices.reshape(-1)
    val = inputs.reshape(-1)
    out = _unpool_scatter(idx, val)
    return out[:N].reshape(SHAPE)


# speedup vs baseline: 5.5644x; 5.5644x over previous
import jax
import jax.numpy as jnp
from jax import lax
from jax.experimental import pallas as pl
from jax.experimental.pallas import tpu as pltpu
from jax.experimental.pallas import tpu_sc as plsc

SHAPE = (4, 224, 224, 96)
N = 19267584
U = 4816896
NCHUNK = 20
ROUNDS = 10
CHUNK = 983040
ACC = CHUNK
NOUT = NCHUNK * CHUNK
PER_CTILE = U // 16
P = 18816
NPIECE = PER_CTILE // P
GROUPS = P // 16
ZB = 2048
ZSHARE = ACC // 16
ZNB = ZSHARE // ZB
DUMP_TW = CHUNK // 16
SENT = -2147483648


def _sc_body(idx_hbm, val_hbm, out_hbm, idx_v, val_v, lidx_v, zero_v, acc):
    c = lax.axis_index("c")
    s = lax.axis_index("s")

    def zinit(i, _):
        zero_v[pl.ds(i * 16, 16)] = jnp.zeros((16,), jnp.float32)
        return 0
    lax.fori_loop(0, ZB // 16, zinit, 0)

    def rnd(r, _):
        q = c * ROUNDS + r
        base = q * CHUNK

        def zblk(i, _):
            pltpu.sync_copy(zero_v, acc.at[pl.ds(s * ZSHARE + i * ZB, ZB)])
            return 0
        lax.fori_loop(0, ZNB, zblk, 0)
        plsc.subcore_barrier()

        def piece(p, _):
            off = s * PER_CTILE + p * P
            pltpu.sync_copy(idx_hbm.at[pl.ds(off, P)], idx_v)
            pltpu.sync_copy(val_hbm.at[pl.ds(off, P)], val_v)

            def grp(i, _):
                iv = idx_v[pl.ds(i * 16, 16)]
                lidx = iv - base
                oob = plsc.bitcast(lidx, jnp.uint32) >= jnp.uint32(CHUNK)
                lidx_v[pl.ds(i * 16, 16)] = jnp.where(oob, jnp.int32(SENT),
                                                      lidx)
                return 0
            lax.fori_loop(0, GROUPS, grp, 0)
            pltpu.sync_copy(
                val_v, acc.at[plsc.Indices(lidx_v, ignored_value=SENT)],
                add=True)
            return 0
        lax.fori_loop(0, NPIECE, piece, 0)
        plsc.subcore_barrier()

        pltpu.sync_copy(acc.at[pl.ds(s * DUMP_TW, DUMP_TW)],
                        out_hbm.at[pl.ds(base + s * DUMP_TW, DUMP_TW)])
        plsc.subcore_barrier()
        return 0
    lax.fori_loop(0, ROUNDS, rnd, 0)


@jax.jit
def _unpool_scatter(idx, val):
    mesh = plsc.VectorSubcoreMesh(core_axis_name="c", subcore_axis_name="s")
    f = pl.kernel(
        _sc_body,
        out_type=jax.ShapeDtypeStruct((NOUT,), jnp.float32),
        mesh=mesh,
        scratch_types=[
            pltpu.VMEM((P,), jnp.int32),
            pltpu.VMEM((P,), jnp.float32),
            pltpu.VMEM((P,), jnp.int32),
            pltpu.VMEM((ZB,), jnp.float32),
            pltpu.VMEM_SHARED((ACC,), jnp.float32),
        ],
    )
    return f(idx, val)


def kernel(inputs, indices, shape_before):
    del shape_before
    idx = indices.reshape(-1)
    val = inputs.reshape(-1)
    out = _unpool_scatter(idx, val)
    return out[:N].reshape(SHAPE)
